# baseline (speedup 1.0000x reference)
import functools

import jax
import jax.numpy as jnp
from jax import lax
from jax.experimental import pallas as pl
from jax.experimental.pallas import tpu as pltpu
from jax.experimental.pallas import tpu_sc as plsc

NC = 2
NS = 16
L = 16
NW = NC * NS

B = 16384
D = 64
BPW = B // NW
CH = 16
NCH = BPW // CH
NSLOT = 3
NB = BPW // 128


def _ring(fire, drain, compute):
    for q in range(NSLOT):
        fire(q, q)

    def tbody(t, carry):
        c0 = NSLOT * t
        for q in range(NSLOT):
            drain(q)
            compute(c0 + q, q)
            fire(c0 + q + NSLOT, q)
        return carry

    lax.fori_loop(0, NCH // NSLOT, tbody, 0)
    for q in range(NCH % NSLOT):
        drain(q)
        compute((NCH // NSLOT) * NSLOT + q, q)


def _item_body(iidx_hbm, iidx2_hbm, itab_hbm, ubias_hbm, ibias_hbm,
               uidx2_hbm, irows_hbm, bias_hbm,
               iidx_v, iidx2_v, uidx2_v, iblk_v, ibuf_v, irows_v, ub_v, ib_v,
               sem0, sem1, sem2, bsem):
    wid = lax.axis_index("s") * NC + lax.axis_index("c")
    pltpu.sync_copy(iidx_hbm.at[wid], iidx_v)
    pltpu.sync_copy(iidx2_hbm.at[wid], iidx2_v)
    pltpu.sync_copy(uidx2_hbm.at[wid], uidx2_v)

    for q in range(NB):
        pltpu.async_copy(ubias_hbm.at[uidx2_v.at[q]],
                         ub_v.at[pl.ds(q * 128, 128)], bsem)
        pltpu.async_copy(ibias_hbm.at[iidx2_v.at[q]],
                         ib_v.at[pl.ds(q * 128, 128)], bsem)

    for c in range(NCH):
        sl = pl.ds(0, L)
        iblk_v[c, sl] = iidx_v[c, sl] >> 3

    sems = (sem0, sem1, sem2)
    lane = lax.iota(jnp.int32, L)
    sl16 = pl.ds(0, L)

    def fire(c, s):
        @pl.when(c < NCH)
        def _():
            ivec = iblk_v[c, sl16]
            for l in range(L):
                pltpu.async_copy(itab_hbm.at[ivec[l]],
                                 ibuf_v.at[s].at[l], sems[s])

    def drain(s):
        pltpu.make_async_copy(itab_hbm.at[pl.ds(0, CH)], ibuf_v.at[s],
                              sems[s]).wait()

    def compute(c, s):
        isub = iidx_v[c, sl16] & 7
        rows = c * CH + lane
        for k in range(D):
            kk = jnp.full((L,), k, jnp.int32)
            v = plsc.load_gather(ibuf_v.at[s], [lane, isub, kk])
            plsc.store_scatter(irows_v, [rows, kk], v)

    _ring(fire, drain, compute)

    pltpu.make_async_copy(ubias_hbm.at[pl.ds(0, BPW)], ub_v, bsem).wait()
    pltpu.make_async_copy(ibias_hbm.at[pl.ds(0, BPW)], ib_v, bsem).wait()
    for j in range(NCH):
        csl = pl.ds(j * CH, L)
        ub_v[csl] = ub_v[csl] + ib_v[csl]

    pltpu.sync_copy(irows_v, irows_hbm.at[wid])
    pltpu.sync_copy(ub_v, bias_hbm.at[wid])


def _user_body(uidx_hbm, utab_hbm, irows_hbm, bias_hbm, out_hbm,
               uidx_v, ublk_v, ubuf_v, irows_v, bias_v, res_v,
               sem0, sem1, sem2, isem):
    wid = lax.axis_index("s") * NC + lax.axis_index("c")
    pltpu.sync_copy(uidx_hbm.at[wid], uidx_v)
    icopy = pltpu.async_copy(irows_hbm.at[wid], irows_v, isem)
    bcopy = pltpu.async_copy(bias_hbm.at[wid], bias_v, isem)

    for c in range(NCH):
        sl = pl.ds(0, L)
        ublk_v[c, sl] = uidx_v[c, sl] >> 3

    sems = (sem0, sem1, sem2)
    lane = lax.iota(jnp.int32, L)
    sl16 = pl.ds(0, L)

    def fire(c, s):
        @pl.when(c < NCH)
        def _():
            uvec = ublk_v[c, sl16]
            for l in range(L):
                pltpu.async_copy(utab_hbm.at[uvec[l]],
                                 ubuf_v.at[s].at[l], sems[s])

    def drain(s):
        pltpu.make_async_copy(utab_hbm.at[pl.ds(0, CH)], ubuf_v.at[s],
                              sems[s]).wait()

    icopy.wait()
    bcopy.wait()

    def compute(c, s):
        usub = uidx_v[c, sl16] & 7
        rows = c * CH + lane
        csl = pl.ds(c * CH, L)
        acc = bias_v[csl]
        for k in range(D):
            kk = jnp.full((L,), k, jnp.int32)
            u = plsc.load_gather(ubuf_v.at[s], [lane, usub, kk])
            v = plsc.load_gather(irows_v, [rows, kk])
            acc = acc + u * v
        res_v[csl] = acc

    _ring(fire, drain, compute)

    pltpu.sync_copy(res_v, out_hbm.at[wid])


@jax.jit
def _mf(user_indices, item_indices, user_embedding, item_embedding,
        user_bias, item_bias):
    uidx = user_indices.astype(jnp.int32).reshape(NW, NCH, CH)
    iidx = item_indices.astype(jnp.int32).reshape(NW, NCH, CH)
    uidx2 = user_indices.astype(jnp.int32).reshape(NW, NB, 128)
    iidx2 = item_indices.astype(jnp.int32).reshape(NW, NB, 128)
    ut = user_embedding.reshape(-1, 8, D)
    it = item_embedding.reshape(-1, 8, D)
    ub = user_bias.reshape(-1)
    ib = item_bias.reshape(-1)

    mesh = plsc.VectorSubcoreMesh(core_axis_name="c", subcore_axis_name="s")
    params = pltpu.CompilerParams(
        needs_layout_passes=False, use_tc_tiling_on_sc=True)

    stage1 = pl.kernel(
        _item_body,
        out_type=(jax.ShapeDtypeStruct((NW, BPW, D), jnp.float32),
                  jax.ShapeDtypeStruct((NW, BPW), jnp.float32)),
        mesh=mesh,
        compiler_params=params,
        scratch_types=[
            pltpu.VMEM((NCH, CH), jnp.int32),
            pltpu.VMEM((NB, 128), jnp.int32),
            pltpu.VMEM((NB, 128), jnp.int32),
            pltpu.VMEM((NCH, CH), jnp.int32),
            pltpu.VMEM((NSLOT, CH, 8, D), jnp.float32),
            pltpu.VMEM((BPW, D), jnp.float32),
            pltpu.VMEM((BPW,), jnp.float32),
            pltpu.VMEM((BPW,), jnp.float32),
            pltpu.SemaphoreType.DMA,
            pltpu.SemaphoreType.DMA,
            pltpu.SemaphoreType.DMA,
            pltpu.SemaphoreType.DMA,
        ],
    )
    irows, bias = stage1(iidx, iidx2, it, ub, ib, uidx2)

    stage2 = pl.kernel(
        _user_body,
        out_type=jax.ShapeDtypeStruct((NW, BPW), jnp.float32),
        mesh=mesh,
        compiler_params=params,
        scratch_types=[
            pltpu.VMEM((NCH, CH), jnp.int32),
            pltpu.VMEM((NCH, CH), jnp.int32),
            pltpu.VMEM((NSLOT, CH, 8, D), jnp.float32),
            pltpu.VMEM((BPW, D), jnp.float32),
            pltpu.VMEM((BPW,), jnp.float32),
            pltpu.VMEM((BPW,), jnp.float32),
            pltpu.SemaphoreType.DMA,
            pltpu.SemaphoreType.DMA,
            pltpu.SemaphoreType.DMA,
            pltpu.SemaphoreType.DMA,
        ],
    )
    out = stage2(uidx, ut, irows, bias)
    return out.reshape(B)


def kernel(user_indices, item_indices, user_embedding, item_embedding,
           user_bias, item_bias):
    return _mf(user_indices, item_indices, user_embedding, item_embedding,
               user_bias, item_bias)


# speedup vs baseline: 1.1248x; 1.1248x over previous
import functools

import jax
import jax.numpy as jnp
from jax import lax
from jax.experimental import pallas as pl
from jax.experimental.pallas import tpu as pltpu
from jax.experimental.pallas import tpu_sc as plsc

NC = 2
NS = 16
L = 16
NW = NC * NS

B = 16384
D = 64
BPW = B // NW
CH = 16
NCH = BPW // CH
GPC = CH // L


def _mf_body(uidx_hbm, iidx_hbm, uidx2_hbm, iidx2_hbm, utab_hbm, itab_hbm,
             ubias_hbm, ibias_hbm, out_hbm, uidx_v, iidx_v, uidx2_v, iidx2_v,
             ublk_v, iblk_v, ubuf_v, ibuf_v, ub_v, ib_v, res_v,
             sem0, sem1, sem2, bsem):
    wid = lax.axis_index("s") * NC + lax.axis_index("c")

    pltpu.sync_copy(uidx_hbm.at[wid], uidx_v)
    pltpu.sync_copy(iidx_hbm.at[wid], iidx_v)
    pltpu.sync_copy(uidx2_hbm.at[wid], uidx2_v)
    pltpu.sync_copy(iidx2_hbm.at[wid], iidx2_v)

    for q in range(BPW // 128):
        pltpu.async_copy(ubias_hbm.at[uidx2_v.at[q]],
                         ub_v.at[pl.ds(q * 128, 128)], bsem)
        pltpu.async_copy(ibias_hbm.at[iidx2_v.at[q]],
                         ib_v.at[pl.ds(q * 128, 128)], bsem)

    for c in range(NCH):
        for j in range(GPC):
            sl = pl.ds(j * L, L)
            ublk_v[c, sl] = uidx_v[c, sl] >> 3
            iblk_v[c, sl] = iidx_v[c, sl] >> 3

    sems = (sem0, sem1, sem2)

    lane = lax.iota(jnp.int32, L)
    sl16 = pl.ds(0, L)

    def fire(c, s):
        @pl.when(c < NCH)
        def _():
            uvec = ublk_v[c, sl16]
            ivec = iblk_v[c, sl16]
            for l in range(L):
                pltpu.async_copy(utab_hbm.at[uvec[l]],
                                 ubuf_v.at[s].at[l], sems[s])
                pltpu.async_copy(itab_hbm.at[ivec[l]],
                                 ibuf_v.at[s].at[l], sems[s])

    def drain(s):
        pltpu.make_async_copy(utab_hbm.at[pl.ds(0, CH)], ubuf_v.at[s],
                              sems[s]).wait()
        pltpu.make_async_copy(itab_hbm.at[pl.ds(0, CH)], ibuf_v.at[s],
                              sems[s]).wait()

    def compute(c, s):
        usub = uidx_v[c, sl16] & 7
        isub = iidx_v[c, sl16] & 7
        csl = pl.ds(c * CH, L)
        acc = ub_v[csl] + ib_v[csl]
        for k in range(D):
            kk = jnp.full((L,), k, jnp.int32)
            u = plsc.load_gather(ubuf_v.at[s], [lane, usub, kk])
            v = plsc.load_gather(ibuf_v.at[s], [lane, isub, kk])
            acc = acc + u * v
        res_v[pl.ds(c * CH, L)] = acc

    pltpu.make_async_copy(ubias_hbm.at[pl.ds(0, BPW)], ub_v, bsem).wait()
    pltpu.make_async_copy(ibias_hbm.at[pl.ds(0, BPW)], ib_v, bsem).wait()

    NSLOT = 3
    for q in range(NSLOT):
        fire(q, q)

    def tbody(t, carry):
        c0 = NSLOT * t
        for q in range(NSLOT):
            drain(q)
            compute(c0 + q, q)
            fire(c0 + q + NSLOT, q)
        return carry

    lax.fori_loop(0, NCH // NSLOT, tbody, 0)

    for q in range(NCH % NSLOT):
        drain(q)
        compute((NCH // NSLOT) * NSLOT + q, q)

    pltpu.sync_copy(res_v, out_hbm.at[wid])


@jax.jit
def _mf(user_indices, item_indices, user_embedding, item_embedding,
        user_bias, item_bias):
    uidx = user_indices.astype(jnp.int32).reshape(NW, NCH, CH)
    iidx = item_indices.astype(jnp.int32).reshape(NW, NCH, CH)
    uidx2 = user_indices.astype(jnp.int32).reshape(NW, BPW // 128, 128)
    iidx2 = item_indices.astype(jnp.int32).reshape(NW, BPW // 128, 128)
    ut = user_embedding.reshape(-1, 8, D)
    it = item_embedding.reshape(-1, 8, D)
    ub = user_bias.reshape(-1)
    ib = item_bias.reshape(-1)

    mesh = plsc.VectorSubcoreMesh(core_axis_name="c", subcore_axis_name="s")
    run = pl.kernel(
        _mf_body,
        out_type=jax.ShapeDtypeStruct((NW, BPW), jnp.float32),
        mesh=mesh,
        compiler_params=pltpu.CompilerParams(
            needs_layout_passes=False, use_tc_tiling_on_sc=True),
        scratch_types=[
            pltpu.VMEM((NCH, CH), jnp.int32),
            pltpu.VMEM((NCH, CH), jnp.int32),
            pltpu.VMEM((BPW // 128, 128), jnp.int32),
            pltpu.VMEM((BPW // 128, 128), jnp.int32),
            pltpu.VMEM((NCH, CH), jnp.int32),
            pltpu.VMEM((NCH, CH), jnp.int32),
            pltpu.VMEM((3, CH, 8, D), jnp.float32),
            pltpu.VMEM((3, CH, 8, D), jnp.float32),
            pltpu.VMEM((BPW,), jnp.float32),
            pltpu.VMEM((BPW,), jnp.float32),
            pltpu.VMEM((BPW,), jnp.float32),
            pltpu.SemaphoreType.DMA,
            pltpu.SemaphoreType.DMA,
            pltpu.SemaphoreType.DMA,
            pltpu.SemaphoreType.DMA,
        ],
    )
    out = run(uidx, iidx, uidx2, iidx2, ut, it, ub, ib)
    return out.reshape(B)


def kernel(user_indices, item_indices, user_embedding, item_embedding,
           user_bias, item_bias):
    return _mf(user_indices, item_indices, user_embedding, item_embedding,
               user_bias, item_bias)


# speedup vs baseline: 1.1365x; 1.0104x over previous
import functools

import jax
import jax.numpy as jnp
from jax import lax
from jax.experimental import pallas as pl
from jax.experimental.pallas import tpu as pltpu
from jax.experimental.pallas import tpu_sc as plsc

NC = 2
NS = 16
L = 16
NW = NC * NS

B = 16384
D = 64
BPW = B // NW
CH = 16
NCH = BPW // CH
GPC = CH // L


def _mf_body(uidx_hbm, iidx_hbm, utab_hbm, itab_hbm,
             ubias_hbm, ibias_hbm, out_hbm, uidx_v, iidx_v,
             ublk_v, iblk_v, ubuf_v, ibuf_v, ub_v, ib_v, res_v,
             sem0, sem1, sem2, bsem0, bsem1, bsem2):
    wid = lax.axis_index("s") * NC + lax.axis_index("c")

    pltpu.sync_copy(uidx_hbm.at[wid], uidx_v)
    pltpu.sync_copy(iidx_hbm.at[wid], iidx_v)
    for c in range(NCH):
        for j in range(GPC):
            sl = pl.ds(j * L, L)
            ublk_v[c, sl] = uidx_v[c, sl] >> 3
            iblk_v[c, sl] = iidx_v[c, sl] >> 3

    sems = (sem0, sem1, sem2)
    bsems = (bsem0, bsem1, bsem2)

    lane = lax.iota(jnp.int32, L)
    sl16 = pl.ds(0, L)

    def fire(c, s):
        @pl.when(c < NCH)
        def _():
            uvec = ublk_v[c, sl16]
            ivec = iblk_v[c, sl16]
            for l in range(L):
                pltpu.async_copy(utab_hbm.at[uvec[l]],
                                 ubuf_v.at[s].at[l], sems[s])
                pltpu.async_copy(itab_hbm.at[ivec[l]],
                                 ibuf_v.at[s].at[l], sems[s])
            pltpu.async_copy(ubias_hbm.at[uidx_v.at[c]], ub_v.at[s], bsems[s])
            pltpu.async_copy(ibias_hbm.at[iidx_v.at[c]], ib_v.at[s], bsems[s])

    def drain(s):
        pltpu.make_async_copy(utab_hbm.at[pl.ds(0, CH)], ubuf_v.at[s],
                              sems[s]).wait()
        pltpu.make_async_copy(itab_hbm.at[pl.ds(0, CH)], ibuf_v.at[s],
                              sems[s]).wait()
        pltpu.make_async_copy(ubias_hbm.at[pl.ds(0, CH)], ub_v.at[s],
                              bsems[s]).wait()
        pltpu.make_async_copy(ibias_hbm.at[pl.ds(0, CH)], ib_v.at[s],
                              bsems[s]).wait()

    def compute(c, s):
        usub = uidx_v[c, sl16] & 7
        isub = iidx_v[c, sl16] & 7
        acc = ub_v[s, sl16] + ib_v[s, sl16]
        for k in range(D):
            kk = jnp.full((L,), k, jnp.int32)
            u = plsc.load_gather(ubuf_v.at[s], [lane, usub, kk])
            v = plsc.load_gather(ibuf_v.at[s], [lane, isub, kk])
            acc = acc + u * v
        res_v[pl.ds(c * CH, L)] = acc

    NSLOT = 3
    for q in range(NSLOT):
        fire(q, q)

    def tbody(t, carry):
        c0 = NSLOT * t
        for q in range(NSLOT):
            drain(q)
            compute(c0 + q, q)
            fire(c0 + q + NSLOT, q)
        return carry

    lax.fori_loop(0, NCH // NSLOT, tbody, 0)

    for q in range(NCH % NSLOT):
        drain(q)
        compute((NCH // NSLOT) * NSLOT + q, q)

    pltpu.sync_copy(res_v, out_hbm.at[wid])


@jax.jit
def _mf(user_indices, item_indices, user_embedding, item_embedding,
        user_bias, item_bias):
    uidx = user_indices.astype(jnp.int32).reshape(NW, NCH, CH)
    iidx = item_indices.astype(jnp.int32).reshape(NW, NCH, CH)
    ut = user_embedding.reshape(-1, 8, D)
    it = item_embedding.reshape(-1, 8, D)
    ub = user_bias.reshape(-1)
    ib = item_bias.reshape(-1)

    mesh = plsc.VectorSubcoreMesh(core_axis_name="c", subcore_axis_name="s")
    run = pl.kernel(
        _mf_body,
        out_type=jax.ShapeDtypeStruct((NW, BPW), jnp.float32),
        mesh=mesh,
        compiler_params=pltpu.CompilerParams(
            needs_layout_passes=False, use_tc_tiling_on_sc=True),
        scratch_types=[
            pltpu.VMEM((NCH, CH), jnp.int32),
            pltpu.VMEM((NCH, CH), jnp.int32),
            pltpu.VMEM((NCH, CH), jnp.int32),
            pltpu.VMEM((NCH, CH), jnp.int32),
            pltpu.VMEM((3, CH, 8, D), jnp.float32),
            pltpu.VMEM((3, CH, 8, D), jnp.float32),
            pltpu.VMEM((3, CH), jnp.float32),
            pltpu.VMEM((3, CH), jnp.float32),
            pltpu.VMEM((BPW,), jnp.float32),
            pltpu.SemaphoreType.DMA,
            pltpu.SemaphoreType.DMA,
            pltpu.SemaphoreType.DMA,
            pltpu.SemaphoreType.DMA,
            pltpu.SemaphoreType.DMA,
            pltpu.SemaphoreType.DMA,
        ],
    )
    out = run(uidx, iidx, ut, it, ub, ib)
    return out.reshape(B)


def kernel(user_indices, item_indices, user_embedding, item_embedding,
           user_bias, item_bias):
    return _mf(user_indices, item_indices, user_embedding, item_embedding,
               user_bias, item_bias)
